# baseline (speedup 1.0000x reference)
import math

import jax
import jax.numpy as jnp
from jax import lax
from jax.experimental import pallas as pl
from jax.experimental.pallas import tpu as pltpu
from jax.experimental.pallas import tpu_sc as plsc

_EMB = 32
_SCALE = float(math.sqrt(_EMB))
_VOCAB = 1000000

_NC = 2
_NS = 16
_NW = _NC * _NS

_BATCH = 16384
_SEQ = 50
_BT = _BATCH // 128
_BT_PER_W = _BT // _NW
_ROWS_PER_W = 128 * _BT_PER_W
_CHUNKS = _BT_PER_W * _SEQ
_NBUF = 4
_T = _CHUNKS // _NBUF

_SEG = 1 << 18
_TCGRID = _SEG // 128


def _tc_convert_body(i0, i1, i2, i3, o_ref):
    for q, i_ref in enumerate((i0, i1, i2, i3)):
        o_ref[:, pl.ds(q * _EMB, _EMB)] = i_ref[...].T * _SCALE


def _sc_body(tok_hbm, table_hbm, out_hbm, idx_all, *bufs):
    cbuf = bufs[:_NBUF]
    rows = bufs[_NBUF:2 * _NBUF]
    obuf = bufs[2 * _NBUF:3 * _NBUF]
    gsem = bufs[3 * _NBUF:4 * _NBUF]
    ssem = bufs[4 * _NBUF:5 * _NBUF]

    wid = lax.axis_index("s") * _NC + lax.axis_index("c")
    pltpu.sync_copy(
        tok_hbm.at[:, pl.ds(wid * _ROWS_PER_W, _ROWS_PER_W)], idx_all)

    iota = jnp.arange(16, dtype=jnp.int32)

    def start_gather(g, b):
        btl = g // _SEQ
        s_ = g % _SEQ
        for j in range(8):
            seg = idx_all[s_, pl.ds(btl * 128 + j * 16, 16)]
            cbuf[b][pl.ds(j * 16, 16)] = lax.bitwise_and(seg, _SEG - 1)
        pltpu.async_copy(table_hbm.at[cbuf[b]], rows[b], gsem[b])

    for b in range(_NBUF):
        start_gather(b, b)

    def outer(t, carry):
        for b in range(_NBUF):
            g = t * _NBUF + b
            pltpu.make_async_copy(
                table_hbm.at[cbuf[b]], rows[b], gsem[b]).wait()

            @pl.when(t > 0)
            def _wait_store():
                pltpu.make_async_copy(
                    obuf[b].at[:, :, pl.ds(0, 128)],
                    out_hbm.at[0, pl.ds(0, 4), 0], ssem[b]).wait()

            btl = g // _SEQ
            s_ = g % _SEQ

            egid = lax.shift_right_logical(iota, 3)
            e8id = lax.bitwise_and(iota, 7)

            def tsc(j, c2):
                base = j * 16
                toks = idx_all[s_, pl.ds(btl * 128 + base, 16)]
                q32v = lax.shift_left(lax.shift_right_logical(toks, 18), 5)
                for k in range(16):
                    r = base + k
                    q32 = q32v[k]
                    rid = jnp.full((16,), 0, jnp.int32) + r
                    v0 = rows[b][r, pl.ds(q32, 16)]
                    plsc.store_scatter(obuf[b], [egid, e8id, rid], v0)
                    v1 = rows[b][r, pl.ds(q32 + 16, 16)]
                    plsc.store_scatter(obuf[b], [egid + 2, e8id, rid], v1)
                return c2

            lax.fori_loop(0, 8, tsc, 0)

            btg = wid * _BT_PER_W + btl
            pltpu.async_copy(
                obuf[b].at[:, :, pl.ds(0, 128)],
                out_hbm.at[s_, pl.ds(0, 4), btg], ssem[b])

            @pl.when(g + _NBUF < _CHUNKS)
            def _next():
                start_gather(g + _NBUF, b)
        return carry

    lax.fori_loop(0, _T, outer, 0)

    for b in range(_NBUF):
        pltpu.make_async_copy(
            obuf[b].at[:, :, pl.ds(0, 128)],
            out_hbm.at[0, pl.ds(0, 4), 0], ssem[b]).wait()


def kernel(tokens, table):
    tok_t = jnp.swapaxes(tokens, 0, 1)
    tbl_t = jnp.swapaxes(table, 0, 1)

    tbl_lin = pl.pallas_call(
        _tc_convert_body,
        grid=(_TCGRID,),
        in_specs=[
            pl.BlockSpec((_EMB, 128), lambda i, q=q: (0, q * _TCGRID + i))
            for q in range(4)
        ],
        out_specs=pl.BlockSpec((128, 128), lambda i: (i, 0)),
        out_shape=jax.ShapeDtypeStruct((_SEG, 128), jnp.float32),
    )(tbl_t, tbl_t, tbl_t, tbl_t)

    mesh = plsc.VectorSubcoreMesh(
        core_axis_name="c", subcore_axis_name="s",
        num_cores=_NC, num_subcores=_NS,
    )
    out5 = pl.kernel(
        _sc_body,
        out_type=jax.ShapeDtypeStruct((_SEQ, 4, _BT, 8, 128), jnp.float32),
        mesh=mesh,
        compiler_params=pltpu.CompilerParams(
            use_tc_tiling_on_sc=False, needs_layout_passes=False),
        scratch_types=(
            [pltpu.VMEM((_SEQ, _ROWS_PER_W), jnp.int32)]
            + [pltpu.VMEM((128,), jnp.int32)] * _NBUF
            + [pltpu.VMEM((128, 128), jnp.float32)] * _NBUF
            + [pltpu.VMEM((4, 8, 144), jnp.float32)] * _NBUF
            + [pltpu.SemaphoreType.DMA] * (2 * _NBUF)
        ),
    )(tok_t, tbl_lin)
    return out5.transpose(2, 4, 0, 1, 3).reshape(_BATCH, _SEQ, _EMB)


# speedup vs baseline: 1.9178x; 1.9178x over previous
import math

import jax
import jax.numpy as jnp
from jax import lax
from jax.experimental import pallas as pl
from jax.experimental.pallas import tpu as pltpu
from jax.experimental.pallas import tpu_sc as plsc

_EMB = 32
_SCALE = float(math.sqrt(_EMB))

_NC = 2
_NS = 16
_NW = _NC * _NS

_BATCH = 16384
_SEQ = 50
_BT = _BATCH // 128
_BT_PER_W = _BT // _NW
_ROWS_PER_W = 128 * _BT_PER_W
_CHUNKS = _BT_PER_W * _SEQ
_NBUF = 4
_T = _CHUNKS // _NBUF


def _body(tok_hbm, table_hbm, out_hbm, idx_all, *bufs):
    rows = bufs[:_NBUF]
    obuf = bufs[_NBUF:2 * _NBUF]
    gsem = bufs[2 * _NBUF:3 * _NBUF]
    ssem = bufs[3 * _NBUF:4 * _NBUF]

    wid = lax.axis_index("s") * _NC + lax.axis_index("c")
    pltpu.sync_copy(
        tok_hbm.at[:, pl.ds(wid * _ROWS_PER_W, _ROWS_PER_W)], idx_all)

    iota = jnp.arange(16, dtype=jnp.int32)

    def idx_slice(g):
        btl = g // _SEQ
        s_ = g % _SEQ
        return idx_all.at[s_, pl.ds(btl * 128, 128)]

    def start_gather(g, b):
        pltpu.async_copy(table_hbm.at[idx_slice(g)], rows[b], gsem[b])

    for b in range(_NBUF):
        start_gather(b, b)

    def outer(t, carry):
        for b in range(_NBUF):
            g = t * _NBUF + b
            pltpu.make_async_copy(
                table_hbm.at[idx_slice(g)], rows[b], gsem[b]).wait()

            @pl.when(t > 0)
            def _wait_store():
                pltpu.make_async_copy(
                    obuf[b].at[:, :, pl.ds(0, 128)],
                    out_hbm.at[0, pl.ds(0, 4), 0], ssem[b]).wait()

            egid = lax.shift_right_logical(iota, 3)
            e8id = lax.bitwise_and(iota, 7)

            def tsc(j, c2):
                base = j * 8
                for k in range(8):
                    r = base + k
                    rid = jnp.full((16,), 0, jnp.int32) + r
                    v0 = rows[b][r, pl.ds(0, 16)] * _SCALE
                    plsc.store_scatter(obuf[b], [egid, e8id, rid], v0)
                    v1 = rows[b][r, pl.ds(16, 16)] * _SCALE
                    plsc.store_scatter(obuf[b], [egid + 2, e8id, rid], v1)
                return c2

            lax.fori_loop(0, 16, tsc, 0)

            btl = g // _SEQ
            s_ = g % _SEQ
            btg = wid * _BT_PER_W + btl
            pltpu.async_copy(
                obuf[b].at[:, :, pl.ds(0, 128)],
                out_hbm.at[s_, pl.ds(0, 4), btg], ssem[b])

            @pl.when(g + _NBUF < _CHUNKS)
            def _next():
                start_gather(g + _NBUF, b)
        return carry

    lax.fori_loop(0, _T, outer, 0)

    for b in range(_NBUF):
        pltpu.make_async_copy(
            obuf[b].at[:, :, pl.ds(0, 128)],
            out_hbm.at[0, pl.ds(0, 4), 0], ssem[b]).wait()


def kernel(tokens, table):
    tok_t = jnp.swapaxes(tokens, 0, 1)
    mesh = plsc.VectorSubcoreMesh(
        core_axis_name="c", subcore_axis_name="s",
        num_cores=_NC, num_subcores=_NS,
    )
    out5 = pl.kernel(
        _body,
        out_type=jax.ShapeDtypeStruct((_SEQ, 4, _BT, 8, 128), jnp.float32),
        mesh=mesh,
        compiler_params=pltpu.CompilerParams(
            use_tc_tiling_on_sc=False, needs_layout_passes=False),
        scratch_types=(
            [pltpu.VMEM((_SEQ, _ROWS_PER_W), jnp.int32)]
            + [pltpu.VMEM((128, _EMB), jnp.float32)] * _NBUF
            + [pltpu.VMEM((4, 8, 144), jnp.float32)] * _NBUF
            + [pltpu.SemaphoreType.DMA] * (2 * _NBUF)
        ),
    )(tok_t, table)
    return out5.transpose(2, 4, 0, 1, 3).reshape(_BATCH, _SEQ, _EMB)
